# Initial kernel scaffold; baseline (speedup 1.0000x reference)
import jax
import jax.numpy as jnp
from jax.experimental import pallas as pl


def kernel(x, W, b):
    raise NotImplementedError("write your pallas kernel here")


# speedup vs baseline: 1.3570x; 1.3570x over previous
import functools

import jax
import jax.numpy as jnp
from jax.experimental import pallas as pl

TOPK = 8
NUM_EXPERTS = 64
BLK = 512


def _router_body(x_ref, wt_ref, b_ref, idx_ref, w_ref):
    x = x_ref[...]
    wt = wt_ref[...]
    logits = jnp.dot(x, wt, preferred_element_type=jnp.float32) + b_ref[...]
    iota = jax.lax.broadcasted_iota(jnp.int32, (BLK, NUM_EXPERTS), 1)

    cur = logits
    vals = []
    idxs = []
    for _ in range(TOPK):
        m = jnp.max(cur, axis=1, keepdims=True)
        idx = jnp.argmax(cur, axis=1).astype(jnp.int32).reshape(BLK, 1)
        vals.append(m)
        idxs.append(idx)
        cur = jnp.where(iota == idx, -jnp.inf, cur)

    v = jnp.concatenate(vals, axis=1)
    e = jnp.exp(v - vals[0])
    w_ref[...] = e / jnp.sum(e, axis=1, keepdims=True)
    idx_ref[...] = jnp.concatenate(idxs, axis=1)


@functools.partial(jax.jit, static_argnames=())
def kernel(x, W, b):
    n = x.shape[0] * x.shape[1]
    d = x.shape[2]
    x_flat = x.reshape(n, d)
    wt = W.T
    b2 = b.reshape(1, NUM_EXPERTS)
    grid = (n // BLK,)
    idx, w = pl.pallas_call(
        _router_body,
        grid=grid,
        in_specs=[
            pl.BlockSpec((BLK, d), lambda i: (i, 0)),
            pl.BlockSpec((d, NUM_EXPERTS), lambda i: (0, 0)),
            pl.BlockSpec((1, NUM_EXPERTS), lambda i: (0, 0)),
        ],
        out_specs=[
            pl.BlockSpec((BLK, TOPK), lambda i: (i, 0)),
            pl.BlockSpec((BLK, TOPK), lambda i: (i, 0)),
        ],
        out_shape=[
            jax.ShapeDtypeStruct((n, TOPK), jnp.int32),
            jax.ShapeDtypeStruct((n, TOPK), jnp.float32),
        ],
    )(x_flat, wt, b2)
    return idx, w


# speedup vs baseline: 1.5425x; 1.1367x over previous
import functools

import jax
import jax.numpy as jnp
from jax.experimental import pallas as pl

TOPK = 8
NUM_EXPERTS = 64
BLK = 1024


def _router_body(x_ref, wt_ref, b_ref, idx_ref, w_ref):
    x = x_ref[...]
    wt = wt_ref[...]
    logits = jnp.dot(x, wt, preferred_element_type=jnp.float32) + b_ref[...]
    iota = jax.lax.broadcasted_iota(jnp.int32, (BLK, NUM_EXPERTS), 1)

    cur = logits
    vals = []
    idxs = []
    for _ in range(TOPK):
        m = jnp.max(cur, axis=1, keepdims=True)
        idx = jnp.argmax(cur, axis=1).astype(jnp.int32).reshape(BLK, 1)
        vals.append(m)
        idxs.append(idx)
        cur = jnp.where(iota == idx, -jnp.inf, cur)

    v = jnp.concatenate(vals, axis=1)
    e = jnp.exp(v - vals[0])
    w_ref[...] = e / jnp.sum(e, axis=1, keepdims=True)
    idx_ref[...] = jnp.concatenate(idxs, axis=1)


@functools.partial(jax.jit, static_argnames=())
def kernel(x, W, b):
    n = x.shape[0] * x.shape[1]
    d = x.shape[2]
    x_flat = x.reshape(n, d)
    wt = W.T
    b2 = b.reshape(1, NUM_EXPERTS)
    grid = (n // BLK,)
    idx, w = pl.pallas_call(
        _router_body,
        grid=grid,
        in_specs=[
            pl.BlockSpec((BLK, d), lambda i: (i, 0)),
            pl.BlockSpec((d, NUM_EXPERTS), lambda i: (0, 0)),
            pl.BlockSpec((1, NUM_EXPERTS), lambda i: (0, 0)),
        ],
        out_specs=[
            pl.BlockSpec((BLK, TOPK), lambda i: (i, 0)),
            pl.BlockSpec((BLK, TOPK), lambda i: (i, 0)),
        ],
        out_shape=[
            jax.ShapeDtypeStruct((n, TOPK), jnp.int32),
            jax.ShapeDtypeStruct((n, TOPK), jnp.float32),
        ],
    )(x_flat, wt, b2)
    return idx, w


# speedup vs baseline: 2.1619x; 1.4016x over previous
import jax
import jax.numpy as jnp
from jax.experimental import pallas as pl
from jax.experimental.pallas import tpu as pltpu
from jax.experimental.pallas import tpu_sc as plsc

TOPK = 8
NE = 64
MM_BLK = 4096
NC, NS, L = 2, 16, 16
NW = NC * NS
CH = 256
N_TOK = 32768

_SORT8 = [(0, 1), (2, 3), (4, 5), (6, 7),
          (0, 2), (1, 3), (4, 6), (5, 7),
          (1, 2), (5, 6), (0, 4), (3, 7),
          (1, 5), (2, 6),
          (1, 4), (3, 6),
          (2, 4), (3, 5),
          (3, 4)]
_BITONIC8 = [(0, 4), (1, 5), (2, 6), (3, 7),
             (0, 2), (1, 3), (4, 6), (5, 7),
             (0, 1), (2, 3), (4, 5), (6, 7)]


def _matmul_t_body(x_ref, w_ref, b_ref, o_ref):
    o_ref[...] = jax.lax.dot_general(
        w_ref[...], x_ref[...],
        (((1,), (1,)), ((), ())),
        preferred_element_type=jnp.float32,
    ) + b_ref[...]


def _ce(v, i, a, b):
    c = v[a] >= v[b]
    va = jnp.where(c, v[a], v[b])
    vb = jnp.where(c, v[b], v[a])
    ia = jnp.where(c, i[a], i[b])
    ib = jnp.where(c, i[b], i[a])
    v[a], v[b], i[a], i[b] = va, vb, ia, ib


def _mk_group(lg_v, ib_v, wb_v):
    def do_group(g, _):
        t = g * L
        run_v = run_i = None
        for eb in range(8):
            cur_v = []
            cur_i = []
            for j in range(8):
                e = eb * 8 + j
                cur_v.append(lg_v[e, pl.ds(t, L)])
                cur_i.append(jnp.full((L,), e, jnp.int32))
            for (a, b) in _SORT8:
                _ce(cur_v, cur_i, a, b)
            if run_v is None:
                run_v, run_i = cur_v, cur_i
            else:
                hi_v, hi_i = [], []
                for k in range(8):
                    cnd = run_v[k] >= cur_v[7 - k]
                    hi_v.append(jnp.where(cnd, run_v[k], cur_v[7 - k]))
                    hi_i.append(jnp.where(cnd, run_i[k], cur_i[7 - k]))
                run_v, run_i = hi_v, hi_i
                for (a, b) in _BITONIC8:
                    _ce(run_v, run_i, a, b)
        es = [jnp.full((L,), 1.0, jnp.float32)]
        for k in range(1, 8):
            es.append(jnp.exp(run_v[k] - run_v[0]))
        tot = es[0]
        for k in range(1, 8):
            tot = tot + es[k]
        for k in range(8):
            wb_v[k, pl.ds(t, L)] = es[k] / tot
            ib_v[k, pl.ds(t, L)] = run_i[k]
        return 0
    return do_group


def _sc_topk_body(lg_hbm, idx_hbm, w_hbm, lg_v0, lg_v1, ib_v, wb_v,
                  sem0, sem1):
    c = jax.lax.axis_index("c")
    s = jax.lax.axis_index("s")
    wid = s * NC + c
    toks_per = N_TOK // NW
    lg_bufs = (lg_v0, lg_v1)
    sems = (sem0, sem1)
    nch = toks_per // CH
    base = wid * toks_per

    copies = [None, None]
    copies[0] = pltpu.async_copy(lg_hbm.at[:, pl.ds(base, CH)],
                                 lg_bufs[0], sems[0])
    for chunk in range(nch):
        cur = chunk % 2
        if chunk + 1 < nch:
            t1 = base + (chunk + 1) * CH
            copies[1 - cur] = pltpu.async_copy(
                lg_hbm.at[:, pl.ds(t1, CH)], lg_bufs[1 - cur], sems[1 - cur])
        copies[cur].wait()
        jax.lax.fori_loop(0, CH // L, _mk_group(lg_bufs[cur], ib_v, wb_v), 0)
        t0 = base + chunk * CH
        pltpu.sync_copy(ib_v, idx_hbm.at[:, pl.ds(t0, CH)])
        pltpu.sync_copy(wb_v, w_hbm.at[:, pl.ds(t0, CH)])


def kernel(x, W, b):
    n = x.shape[0] * x.shape[1]
    d = x.shape[2]
    x_flat = x.reshape(n, d)
    b2 = b.reshape(NE, 1)

    logits_t = pl.pallas_call(
        _matmul_t_body,
        grid=(n // MM_BLK,),
        in_specs=[
            pl.BlockSpec((MM_BLK, d), lambda i: (i, 0)),
            pl.BlockSpec((NE, d), lambda i: (0, 0)),
            pl.BlockSpec((NE, 1), lambda i: (0, 0)),
        ],
        out_specs=pl.BlockSpec((NE, MM_BLK), lambda i: (0, i)),
        out_shape=jax.ShapeDtypeStruct((NE, n), jnp.float32),
    )(x_flat, W, b2)

    topk = pl.kernel(
        _sc_topk_body,
        out_type=(
            jax.ShapeDtypeStruct((TOPK, n), jnp.int32),
            jax.ShapeDtypeStruct((TOPK, n), jnp.float32),
        ),
        mesh=plsc.VectorSubcoreMesh(core_axis_name="c", subcore_axis_name="s",
                                    num_cores=NC, num_subcores=NS),
        scratch_types=[
            pltpu.VMEM((NE, CH), jnp.float32),
            pltpu.VMEM((NE, CH), jnp.float32),
            pltpu.VMEM((TOPK, CH), jnp.int32),
            pltpu.VMEM((TOPK, CH), jnp.float32),
            pltpu.SemaphoreType.DMA,
            pltpu.SemaphoreType.DMA,
        ],
    )
    idx_t, w_t = topk(logits_t)
    return idx_t.T, w_t.T


# speedup vs baseline: 2.1847x; 1.0106x over previous
import jax
import jax.numpy as jnp
from jax.experimental import pallas as pl
from jax.experimental.pallas import tpu as pltpu
from jax.experimental.pallas import tpu_sc as plsc

TOPK = 8
NE = 64
MM_BLK = 4096
NC, NS, L = 2, 16, 16
NW = NC * NS
CH = 512
N_TOK = 32768

_SORT8 = [(0, 1), (2, 3), (4, 5), (6, 7),
          (0, 2), (1, 3), (4, 6), (5, 7),
          (1, 2), (5, 6), (0, 4), (3, 7),
          (1, 5), (2, 6),
          (1, 4), (3, 6),
          (2, 4), (3, 5),
          (3, 4)]
_BITONIC8 = [(0, 4), (1, 5), (2, 6), (3, 7),
             (0, 2), (1, 3), (4, 6), (5, 7),
             (0, 1), (2, 3), (4, 5), (6, 7)]


def _matmul_t_body(x_ref, w_ref, b_ref, o_ref):
    o_ref[...] = jax.lax.dot_general(
        w_ref[...], x_ref[...],
        (((1,), (1,)), ((), ())),
        preferred_element_type=jnp.float32,
    ) + b_ref[...]


def _ce(v, i, a, b):
    c = v[a] >= v[b]
    va = jnp.where(c, v[a], v[b])
    vb = jnp.where(c, v[b], v[a])
    ia = jnp.where(c, i[a], i[b])
    ib = jnp.where(c, i[b], i[a])
    v[a], v[b], i[a], i[b] = va, vb, ia, ib


def _mk_group(lg_v, ib_v, wb_v):
    def do_group(g, _):
        t = g * L
        run_v = run_i = None
        for eb in range(8):
            cur_v = []
            cur_i = []
            for j in range(8):
                e = eb * 8 + j
                cur_v.append(lg_v[e, pl.ds(t, L)])
                cur_i.append(jnp.full((L,), e, jnp.int32))
            for (a, b) in _SORT8:
                _ce(cur_v, cur_i, a, b)
            if run_v is None:
                run_v, run_i = cur_v, cur_i
            else:
                hi_v, hi_i = [], []
                for k in range(8):
                    cnd = run_v[k] >= cur_v[7 - k]
                    hi_v.append(jnp.where(cnd, run_v[k], cur_v[7 - k]))
                    hi_i.append(jnp.where(cnd, run_i[k], cur_i[7 - k]))
                run_v, run_i = hi_v, hi_i
                for (a, b) in _BITONIC8:
                    _ce(run_v, run_i, a, b)
        es = [jnp.full((L,), 1.0, jnp.float32)]
        for k in range(1, 8):
            es.append(jnp.exp(run_v[k] - run_v[0]))
        tot = es[0]
        for k in range(1, 8):
            tot = tot + es[k]
        for k in range(8):
            wb_v[k, pl.ds(t, L)] = es[k] / tot
            ib_v[k, pl.ds(t, L)] = run_i[k]
        return 0
    return do_group


def _sc_topk_body(lg_hbm, idx_hbm, w_hbm, lg_v0, lg_v1, ib_v, wb_v,
                  sem0, sem1):
    c = jax.lax.axis_index("c")
    s = jax.lax.axis_index("s")
    wid = s * NC + c
    toks_per = N_TOK // NW
    lg_bufs = (lg_v0, lg_v1)
    sems = (sem0, sem1)
    nch = toks_per // CH
    base = wid * toks_per

    copies = [None, None]
    copies[0] = pltpu.async_copy(lg_hbm.at[:, pl.ds(base, CH)],
                                 lg_bufs[0], sems[0])
    for chunk in range(nch):
        cur = chunk % 2
        if chunk + 1 < nch:
            t1 = base + (chunk + 1) * CH
            copies[1 - cur] = pltpu.async_copy(
                lg_hbm.at[:, pl.ds(t1, CH)], lg_bufs[1 - cur], sems[1 - cur])
        copies[cur].wait()
        jax.lax.fori_loop(0, CH // L, _mk_group(lg_bufs[cur], ib_v, wb_v), 0)
        t0 = base + chunk * CH
        pltpu.sync_copy(ib_v, idx_hbm.at[:, pl.ds(t0, CH)])
        pltpu.sync_copy(wb_v, w_hbm.at[:, pl.ds(t0, CH)])


def kernel(x, W, b):
    n = x.shape[0] * x.shape[1]
    d = x.shape[2]
    x_flat = x.reshape(n, d)
    b2 = b.reshape(NE, 1)

    logits_t = pl.pallas_call(
        _matmul_t_body,
        grid=(n // MM_BLK,),
        in_specs=[
            pl.BlockSpec((MM_BLK, d), lambda i: (i, 0)),
            pl.BlockSpec((NE, d), lambda i: (0, 0)),
            pl.BlockSpec((NE, 1), lambda i: (0, 0)),
        ],
        out_specs=pl.BlockSpec((NE, MM_BLK), lambda i: (0, i)),
        out_shape=jax.ShapeDtypeStruct((NE, n), jnp.float32),
    )(x_flat, W, b2)

    topk = pl.kernel(
        _sc_topk_body,
        out_type=(
            jax.ShapeDtypeStruct((TOPK, n), jnp.int32),
            jax.ShapeDtypeStruct((TOPK, n), jnp.float32),
        ),
        mesh=plsc.VectorSubcoreMesh(core_axis_name="c", subcore_axis_name="s",
                                    num_cores=NC, num_subcores=NS),
        scratch_types=[
            pltpu.VMEM((NE, CH), jnp.float32),
            pltpu.VMEM((NE, CH), jnp.float32),
            pltpu.VMEM((TOPK, CH), jnp.int32),
            pltpu.VMEM((TOPK, CH), jnp.float32),
            pltpu.SemaphoreType.DMA,
            pltpu.SemaphoreType.DMA,
        ],
    )
    idx_t, w_t = topk(logits_t)
    return idx_t.T, w_t.T


# speedup vs baseline: 2.1859x; 1.0006x over previous
import jax
import jax.numpy as jnp
from jax.experimental import pallas as pl
from jax.experimental.pallas import tpu as pltpu
from jax.experimental.pallas import tpu_sc as plsc

TOPK = 8
NE = 64
MM_BLK = 4096
NC, NS, L = 2, 16, 16
NW = NC * NS
CH = 512
N_TOK = 32768

_SORT8 = [(0, 1), (2, 3), (4, 5), (6, 7),
          (0, 2), (1, 3), (4, 6), (5, 7),
          (1, 2), (5, 6), (0, 4), (3, 7),
          (1, 5), (2, 6),
          (1, 4), (3, 6),
          (2, 4), (3, 5),
          (3, 4)]
_BITONIC8 = [(0, 4), (1, 5), (2, 6), (3, 7),
             (0, 2), (1, 3), (4, 6), (5, 7),
             (0, 1), (2, 3), (4, 5), (6, 7)]


def _matmul_t_body(x_ref, w_ref, b_ref, o_ref):
    o_ref[...] = jax.lax.dot_general(
        w_ref[...], x_ref[...],
        (((1,), (1,)), ((), ())),
        preferred_element_type=jnp.float32,
    ) + b_ref[...]


def _ce(v, i, a, b):
    c = v[a] >= v[b]
    va = jnp.where(c, v[a], v[b])
    vb = jnp.where(c, v[b], v[a])
    ia = jnp.where(c, i[a], i[b])
    ib = jnp.where(c, i[b], i[a])
    v[a], v[b], i[a], i[b] = va, vb, ia, ib


def _mk_group(lg_v, ib_v, wb_v):
    def do_group(g, _):
        t = g * L
        run_v = run_i = None
        for eb in range(8):
            cur_v = []
            cur_i = []
            for j in range(8):
                e = eb * 8 + j
                cur_v.append(lg_v[e, pl.ds(t, L)])
                cur_i.append(jnp.full((L,), e, jnp.int32))
            for (a, b) in _SORT8:
                _ce(cur_v, cur_i, a, b)
            if run_v is None:
                run_v, run_i = cur_v, cur_i
            else:
                hi_v, hi_i = [], []
                for k in range(8):
                    cnd = run_v[k] >= cur_v[7 - k]
                    hi_v.append(jnp.where(cnd, run_v[k], cur_v[7 - k]))
                    hi_i.append(jnp.where(cnd, run_i[k], cur_i[7 - k]))
                run_v, run_i = hi_v, hi_i
                for (a, b) in _BITONIC8:
                    _ce(run_v, run_i, a, b)
        es = [jnp.full((L,), 1.0, jnp.float32)]
        for k in range(1, 8):
            es.append(jnp.exp(run_v[k] - run_v[0]))
        tot = es[0]
        for k in range(1, 8):
            tot = tot + es[k]
        for k in range(8):
            wb_v[k, pl.ds(t, L)] = es[k] / tot
            ib_v[k, pl.ds(t, L)] = run_i[k]
        return 0
    return do_group


def _sc_topk_body(lg_hbm, idx_hbm, w_hbm, lg_v0, lg_v1, ib_v, wb_v,
                  sem0, sem1):
    c = jax.lax.axis_index("c")
    s = jax.lax.axis_index("s")
    wid = s * NC + c
    toks_per = N_TOK // NW
    lg_bufs = (lg_v0, lg_v1)
    sems = (sem0, sem1)
    nch = toks_per // CH
    base = wid * toks_per

    copies = [None, None]
    copies[0] = pltpu.async_copy(lg_hbm.at[:, pl.ds(base, CH)],
                                 lg_bufs[0], sems[0])
    for chunk in range(nch):
        cur = chunk % 2
        if chunk + 1 < nch:
            t1 = base + (chunk + 1) * CH
            copies[1 - cur] = pltpu.async_copy(
                lg_hbm.at[:, pl.ds(t1, CH)], lg_bufs[1 - cur], sems[1 - cur])
        copies[cur].wait()
        jax.lax.fori_loop(0, CH // L, _mk_group(lg_bufs[cur], ib_v, wb_v), 0)
        t0 = base + chunk * CH
        pltpu.sync_copy(ib_v, idx_hbm.at[:, pl.ds(t0, CH)])
        pltpu.sync_copy(wb_v, w_hbm.at[:, pl.ds(t0, CH)])


def kernel(x, W, b):
    n = x.shape[0] * x.shape[1]
    d = x.shape[2]
    x_flat = x.reshape(n, d)
    b2 = b.reshape(NE, 1)

    logits_t = pl.pallas_call(
        _matmul_t_body,
        grid=(n // MM_BLK,),
        compiler_params=pltpu.CompilerParams(
            dimension_semantics=("parallel",)),
        in_specs=[
            pl.BlockSpec((MM_BLK, d), lambda i: (i, 0)),
            pl.BlockSpec((NE, d), lambda i: (0, 0)),
            pl.BlockSpec((NE, 1), lambda i: (0, 0)),
        ],
        out_specs=pl.BlockSpec((NE, MM_BLK), lambda i: (0, i)),
        out_shape=jax.ShapeDtypeStruct((NE, n), jnp.float32),
    )(x_flat, W, b2)

    topk = pl.kernel(
        _sc_topk_body,
        out_type=(
            jax.ShapeDtypeStruct((TOPK, n), jnp.int32),
            jax.ShapeDtypeStruct((TOPK, n), jnp.float32),
        ),
        mesh=plsc.VectorSubcoreMesh(core_axis_name="c", subcore_axis_name="s",
                                    num_cores=NC, num_subcores=NS),
        scratch_types=[
            pltpu.VMEM((NE, CH), jnp.float32),
            pltpu.VMEM((NE, CH), jnp.float32),
            pltpu.VMEM((TOPK, CH), jnp.int32),
            pltpu.VMEM((TOPK, CH), jnp.float32),
            pltpu.SemaphoreType.DMA,
            pltpu.SemaphoreType.DMA,
        ],
    )
    idx_t, w_t = topk(logits_t)
    return idx_t.T, w_t.T
